# baseline (speedup 1.0000x reference)
import functools

import jax
import jax.numpy as jnp
from jax import lax
from jax.experimental import pallas as pl
from jax.experimental.pallas import tpu as pltpu
from jax.experimental.pallas import tpu_sc as plsc

NC = 2
NS = 16
NW = NC * NS
CH = 128
K0 = 55
K1 = 105
KMAX = max(K0, K1)
K_ALLOC = max(16 * (K0 + K1), 16 * K0 + 15 * K1 + KMAX, 15 * K0 + KMAX)


def _zero_copy_chunks(rows_per_tile):
    chunks = []
    r = 0
    while r < rows_per_tile:
        sz = min(CH, rows_per_tile - r)
        chunks.append((r, sz))
        r += sz
    return chunks


@functools.lru_cache(maxsize=None)
def _sc_propagate(n_pad, d, with_gather):
    rows_per_tile = n_pad // NS
    chunks = _zero_copy_chunks(rows_per_tile)
    mesh = plsc.VectorSubcoreMesh(core_axis_name="c", subcore_axis_name="s")
    k_deg = (16 * (K0 + K1)) // NW

    def zero_acc(const_hbm, buf_v, acc_sh, r0):
        pltpu.sync_copy(const_hbm.at[pl.ds(0, CH)], buf_v)
        for (off, sz) in chunks:
            pltpu.sync_copy(buf_v.at[pl.ds(0, sz)],
                            acc_sh.at[pl.ds(r0 + off, sz)])

    def write_back(out_hbm, buf_v, acc_sh, cid, r0):
        for (off, sz) in chunks:
            pltpu.sync_copy(acc_sh.at[pl.ds(r0 + off, sz)],
                            buf_v.at[pl.ds(0, sz)])
            pltpu.sync_copy(buf_v.at[pl.ds(0, sz)],
                            out_hbm.at[cid, pl.ds(r0 + off, sz)])

    def gather_body(g_hbm, edges_hbm, const_hbm, out_hbm,
                    idx_v, rows_v, acc_sh, sem):
        cid = lax.axis_index("c")
        sid = lax.axis_index("s")
        r0 = sid * rows_per_tile
        zero_acc(const_hbm, rows_v, acc_sh, r0)
        plsc.subcore_barrier()
        base = jnp.where(cid == 0, sid * K0, NS * K0 + sid * K1)
        k_c = jnp.where(cid == 0, K0, K1)
        pltpu.sync_copy(edges_hbm.at[pl.ds(base, KMAX)], idx_v)

        @pl.loop(0, k_c)
        def _edge_chunk(j):
            pltpu.async_copy(g_hbm.at[idx_v.at[j, 0]], rows_v, sem).wait()
            pltpu.sync_copy(rows_v, acc_sh.at[idx_v.at[j, 1]], add=True)

        plsc.subcore_barrier()
        write_back(out_hbm, rows_v, acc_sh, cid, r0)

    def deg_body(g_hbm, edges_hbm, const_hbm, out_hbm,
                 idx_v, ones_v, acc_sh):
        cid = lax.axis_index("c")
        sid = lax.axis_index("s")
        wid = cid * NS + sid
        r0 = sid * rows_per_tile
        zero_acc(const_hbm, ones_v, acc_sh, r0)
        plsc.subcore_barrier()
        pltpu.sync_copy(edges_hbm.at[pl.ds(wid * k_deg, k_deg)], idx_v)
        pltpu.sync_copy(const_hbm.at[pl.ds(CH, CH)], ones_v)

        @pl.loop(0, k_deg)
        def _edge_chunk(j):
            pltpu.sync_copy(ones_v, acc_sh.at[idx_v.at[j, 1]], add=True)

        plsc.subcore_barrier()
        write_back(out_hbm, ones_v, acc_sh, cid, r0)

    if with_gather:
        scratch = [
            pltpu.VMEM((KMAX, 2, CH), jnp.int32),
            pltpu.VMEM((CH, d), jnp.float32),
            pltpu.VMEM_SHARED((n_pad, d), jnp.float32),
            pltpu.SemaphoreType.DMA,
        ]
        body = gather_body
    else:
        scratch = [
            pltpu.VMEM((k_deg, 2, CH), jnp.int32),
            pltpu.VMEM((CH, d), jnp.float32),
            pltpu.VMEM_SHARED((n_pad, d), jnp.float32),
        ]
        body = deg_body

    return pl.kernel(
        body,
        out_type=jax.ShapeDtypeStruct((NC, n_pad, d), jnp.float32),
        mesh=mesh,
        scratch_types=scratch,
        compiler_params=pltpu.CompilerParams(use_tc_tiling_on_sc=False),
        name=f"sc_prop_d{d}_{'gather' if with_gather else 'deg'}",
    )


def _dinv(degp_ref):
    deg = degp_ref[0, :, 0:1] + degp_ref[1, :, 0:1] + 1.0
    return lax.rsqrt(deg)


def _tc_pre_body(degp, x, w1, g0):
    dinv = _dinv(degp)
    g0[...] = jnp.dot(x[...], w1[...],
                      preferred_element_type=jnp.float32) * dinv


def _tc_mid_body(degp, s, g, w2, b1, x1_out, g1_out):
    dinv = _dinv(degp)
    x1 = jnp.maximum(dinv * (s[0] + s[1] + g[...]) + b1[...], 0.0)
    x1_out[...] = x1
    g1_out[...] = jnp.dot(x1, w2[...],
                          preferred_element_type=jnp.float32) * dinv


def _tc_jk_body(degp, s, g1, b2, x1, wl1, wl2, gy_out):
    dinv = _dinv(degp)
    x2 = jnp.maximum(dinv * (s[0] + s[1] + g1[...]) + b2[...], 0.0)
    y = (jnp.dot(x1[...], wl1[...], preferred_element_type=jnp.float32)
         + jnp.dot(x2, wl2[...], preferred_element_type=jnp.float32))
    gy_out[...] = y * dinv


def _tc_out_body(degp, s, gy, blin, out):
    dinv = _dinv(degp)
    out[...] = dinv * (s[0] + s[1] + gy[...]) + blin[...]


def kernel(x, edge_index, W1, b1, W2, b2, Wlin, blin):
    n, f = x.shape
    hid = W1.shape[1]
    ncls = Wlin.shape[1]
    e = edge_index.shape[1]
    n_pad = -(-(n + 1) // (NS * 8)) * (NS * 8)
    assert 16 * (K0 + K1) * CH >= e
    e_alloc = K_ALLOC * CH
    rb = 1000
    assert n % rb == 0
    grid = n // rb

    pad = e_alloc - e
    src_p = jnp.concatenate([edge_index[0], jnp.zeros((pad,), jnp.int32)])
    dst_p = jnp.concatenate([edge_index[1], jnp.full((pad,), n, jnp.int32)])
    edges = jnp.stack([src_p.reshape(K_ALLOC, CH),
                       dst_p.reshape(K_ALLOC, CH)], axis=1)

    zeros_h = jnp.zeros((2 * CH, hid), jnp.float32)
    zeros_c = jnp.zeros((2 * CH, ncls), jnp.float32)
    ones16 = jnp.concatenate([jnp.zeros((CH, 16), jnp.float32),
                              jnp.ones((CH, 16), jnp.float32)])
    b1r = b1.reshape(1, hid)
    b2r = b2.reshape(1, hid)
    blinr = blin.reshape(1, ncls)
    wl1 = Wlin[:hid]
    wl2 = Wlin[hid:]

    deg_kernel = _sc_propagate(n_pad, 16, False)
    prop_h = _sc_propagate(n_pad, hid, True)
    prop_c = _sc_propagate(n_pad, ncls, True)

    dummy16 = jnp.zeros((n, 16), jnp.float32)
    degp = deg_kernel(dummy16, edges, ones16)

    degp_spec = pl.BlockSpec((NC, rb, 16), lambda i: (0, i, 0))
    row_spec_h = pl.BlockSpec((rb, hid), lambda i: (i, 0))
    row_spec_c = pl.BlockSpec((rb, ncls), lambda i: (i, 0))
    s_spec_h = pl.BlockSpec((NC, rb, hid), lambda i: (0, i, 0))
    s_spec_c = pl.BlockSpec((NC, rb, ncls), lambda i: (0, i, 0))
    full = lambda shape: pl.BlockSpec(shape, lambda i: tuple(0 for _ in shape))

    g0 = pl.pallas_call(
        _tc_pre_body,
        grid=(grid,),
        in_specs=[degp_spec, pl.BlockSpec((rb, f), lambda i: (i, 0)),
                  full((f, hid))],
        out_specs=row_spec_h,
        out_shape=jax.ShapeDtypeStruct((n, hid), jnp.float32),
    )(degp, x, W1)

    s0 = prop_h(g0, edges, zeros_h)

    x1, g1 = pl.pallas_call(
        _tc_mid_body,
        grid=(grid,),
        in_specs=[degp_spec, s_spec_h, row_spec_h, full((hid, hid)),
                  full((1, hid))],
        out_specs=[row_spec_h, row_spec_h],
        out_shape=[jax.ShapeDtypeStruct((n, hid), jnp.float32),
                   jax.ShapeDtypeStruct((n, hid), jnp.float32)],
    )(degp, s0, g0, W2, b1r)

    s1 = prop_h(g1, edges, zeros_h)

    gy = pl.pallas_call(
        _tc_jk_body,
        grid=(grid,),
        in_specs=[degp_spec, s_spec_h, row_spec_h, full((1, hid)),
                  row_spec_h, full((hid, ncls)), full((hid, ncls))],
        out_specs=row_spec_c,
        out_shape=jax.ShapeDtypeStruct((n, ncls), jnp.float32),
    )(degp, s1, g1, b2r, x1, wl1, wl2)

    s2 = prop_c(gy, edges, zeros_c)

    out = pl.pallas_call(
        _tc_out_body,
        grid=(grid,),
        in_specs=[degp_spec, s_spec_c, row_spec_c, full((1, ncls))],
        out_specs=row_spec_c,
        out_shape=jax.ShapeDtypeStruct((n, ncls), jnp.float32),
    )(degp, s2, gy, blinr)

    return (out, out)


# speedup vs baseline: 1.0742x; 1.0742x over previous
import functools

import jax
import jax.numpy as jnp
from jax import lax
from jax.experimental import pallas as pl
from jax.experimental.pallas import tpu as pltpu
from jax.experimental.pallas import tpu_sc as plsc

NC = 2
NS = 16
NW = NC * NS
CH = 128
K0 = 105
K1 = 55
KMAX = max(K0, K1)
K_ALLOC = max(16 * (K0 + K1), 16 * K0 + 15 * K1 + KMAX, 15 * K0 + KMAX)


def _zero_copy_chunks(rows_per_tile):
    chunks = []
    r = 0
    while r < rows_per_tile:
        sz = min(CH, rows_per_tile - r)
        chunks.append((r, sz))
        r += sz
    return chunks


@functools.lru_cache(maxsize=None)
def _sc_propagate(n_pad, d, with_gather):
    rows_per_tile = n_pad // NS
    chunks = _zero_copy_chunks(rows_per_tile)
    mesh = plsc.VectorSubcoreMesh(core_axis_name="c", subcore_axis_name="s")
    k_deg = (16 * (K0 + K1)) // NW

    def zero_acc(const_hbm, buf_v, acc_sh, r0):
        pltpu.sync_copy(const_hbm.at[pl.ds(0, CH)], buf_v)
        for (off, sz) in chunks:
            pltpu.sync_copy(buf_v.at[pl.ds(0, sz)],
                            acc_sh.at[pl.ds(r0 + off, sz)])

    def write_back(out_hbm, buf_v, acc_sh, cid, r0):
        for (off, sz) in chunks:
            pltpu.sync_copy(acc_sh.at[pl.ds(r0 + off, sz)],
                            buf_v.at[pl.ds(0, sz)])
            pltpu.sync_copy(buf_v.at[pl.ds(0, sz)],
                            out_hbm.at[cid, pl.ds(r0 + off, sz)])

    def gather_body(g_hbm, edges_hbm, const_hbm, out_hbm,
                    idx_v, rows_v, acc_sh, sem):
        cid = lax.axis_index("c")
        sid = lax.axis_index("s")
        r0 = sid * rows_per_tile
        zero_acc(const_hbm, rows_v, acc_sh, r0)
        plsc.subcore_barrier()
        base = jnp.where(cid == 0, sid * K0, NS * K0 + sid * K1)
        k_c = jnp.where(cid == 0, K0, K1)
        pltpu.sync_copy(edges_hbm.at[pl.ds(base, KMAX)], idx_v)

        @pl.loop(0, k_c)
        def _edge_chunk(j):
            pltpu.async_copy(g_hbm.at[idx_v.at[j, 0]], rows_v, sem).wait()
            pltpu.sync_copy(rows_v, acc_sh.at[idx_v.at[j, 1]], add=True)

        plsc.subcore_barrier()
        write_back(out_hbm, rows_v, acc_sh, cid, r0)

    def deg_body(g_hbm, edges_hbm, const_hbm, out_hbm,
                 idx_v, ones_v, acc_sh):
        cid = lax.axis_index("c")
        sid = lax.axis_index("s")
        wid = cid * NS + sid
        r0 = sid * rows_per_tile
        zero_acc(const_hbm, ones_v, acc_sh, r0)
        plsc.subcore_barrier()
        pltpu.sync_copy(edges_hbm.at[pl.ds(wid * k_deg, k_deg)], idx_v)
        pltpu.sync_copy(const_hbm.at[pl.ds(CH, CH)], ones_v)

        @pl.loop(0, k_deg)
        def _edge_chunk(j):
            pltpu.sync_copy(ones_v, acc_sh.at[idx_v.at[j, 1]], add=True)

        plsc.subcore_barrier()
        write_back(out_hbm, ones_v, acc_sh, cid, r0)

    if with_gather:
        scratch = [
            pltpu.VMEM((KMAX, 2, CH), jnp.int32),
            pltpu.VMEM((CH, d), jnp.float32),
            pltpu.VMEM_SHARED((n_pad, d), jnp.float32),
            pltpu.SemaphoreType.DMA,
        ]
        body = gather_body
    else:
        scratch = [
            pltpu.VMEM((k_deg, 2, CH), jnp.int32),
            pltpu.VMEM((CH, d), jnp.float32),
            pltpu.VMEM_SHARED((n_pad, d), jnp.float32),
        ]
        body = deg_body

    return pl.kernel(
        body,
        out_type=jax.ShapeDtypeStruct((NC, n_pad, d), jnp.float32),
        mesh=mesh,
        scratch_types=scratch,
        compiler_params=pltpu.CompilerParams(use_tc_tiling_on_sc=False),
        name=f"sc_prop_d{d}_{'gather' if with_gather else 'deg'}",
    )


def _dinv(degp_ref):
    deg = degp_ref[0, :, 0:1] + degp_ref[1, :, 0:1] + 1.0
    return lax.rsqrt(deg)


def _tc_pre_body(degp, x, w1, g0):
    dinv = _dinv(degp)
    g0[...] = jnp.dot(x[...], w1[...],
                      preferred_element_type=jnp.float32) * dinv


def _tc_mid_body(degp, s, g, w2, b1, x1_out, g1_out):
    dinv = _dinv(degp)
    x1 = jnp.maximum(dinv * (s[0] + s[1] + g[...]) + b1[...], 0.0)
    x1_out[...] = x1
    g1_out[...] = jnp.dot(x1, w2[...],
                          preferred_element_type=jnp.float32) * dinv


def _tc_jk_body(degp, s, g1, b2, x1, wl1, wl2, gy_out):
    dinv = _dinv(degp)
    x2 = jnp.maximum(dinv * (s[0] + s[1] + g1[...]) + b2[...], 0.0)
    y = (jnp.dot(x1[...], wl1[...], preferred_element_type=jnp.float32)
         + jnp.dot(x2, wl2[...], preferred_element_type=jnp.float32))
    gy_out[...] = y * dinv


def _tc_out_body(degp, s, gy, blin, out):
    dinv = _dinv(degp)
    out[...] = dinv * (s[0] + s[1] + gy[...]) + blin[...]


def kernel(x, edge_index, W1, b1, W2, b2, Wlin, blin):
    n, f = x.shape
    hid = W1.shape[1]
    ncls = Wlin.shape[1]
    e = edge_index.shape[1]
    n_pad = -(-(n + 1) // (NS * 8)) * (NS * 8)
    assert 16 * (K0 + K1) * CH >= e
    e_alloc = K_ALLOC * CH
    rb = 1000
    assert n % rb == 0
    grid = n // rb

    pad = e_alloc - e
    src_p = jnp.concatenate([edge_index[0], jnp.zeros((pad,), jnp.int32)])
    dst_p = jnp.concatenate([edge_index[1], jnp.full((pad,), n, jnp.int32)])
    edges = jnp.stack([src_p.reshape(K_ALLOC, CH),
                       dst_p.reshape(K_ALLOC, CH)], axis=1)

    zeros_h = jnp.zeros((2 * CH, hid), jnp.float32)
    zeros_c = jnp.zeros((2 * CH, ncls), jnp.float32)
    ones16 = jnp.concatenate([jnp.zeros((CH, 16), jnp.float32),
                              jnp.ones((CH, 16), jnp.float32)])
    b1r = b1.reshape(1, hid)
    b2r = b2.reshape(1, hid)
    blinr = blin.reshape(1, ncls)
    wl1 = Wlin[:hid]
    wl2 = Wlin[hid:]

    deg_kernel = _sc_propagate(n_pad, 16, False)
    prop_h = _sc_propagate(n_pad, hid, True)
    prop_c = _sc_propagate(n_pad, ncls, True)

    dummy16 = jnp.zeros((n, 16), jnp.float32)
    degp = deg_kernel(dummy16, edges, ones16)

    degp_spec = pl.BlockSpec((NC, rb, 16), lambda i: (0, i, 0))
    row_spec_h = pl.BlockSpec((rb, hid), lambda i: (i, 0))
    row_spec_c = pl.BlockSpec((rb, ncls), lambda i: (i, 0))
    s_spec_h = pl.BlockSpec((NC, rb, hid), lambda i: (0, i, 0))
    s_spec_c = pl.BlockSpec((NC, rb, ncls), lambda i: (0, i, 0))
    full = lambda shape: pl.BlockSpec(shape, lambda i: tuple(0 for _ in shape))

    g0 = pl.pallas_call(
        _tc_pre_body,
        grid=(grid,),
        in_specs=[degp_spec, pl.BlockSpec((rb, f), lambda i: (i, 0)),
                  full((f, hid))],
        out_specs=row_spec_h,
        out_shape=jax.ShapeDtypeStruct((n, hid), jnp.float32),
    )(degp, x, W1)

    s0 = prop_h(g0, edges, zeros_h)

    x1, g1 = pl.pallas_call(
        _tc_mid_body,
        grid=(grid,),
        in_specs=[degp_spec, s_spec_h, row_spec_h, full((hid, hid)),
                  full((1, hid))],
        out_specs=[row_spec_h, row_spec_h],
        out_shape=[jax.ShapeDtypeStruct((n, hid), jnp.float32),
                   jax.ShapeDtypeStruct((n, hid), jnp.float32)],
    )(degp, s0, g0, W2, b1r)

    s1 = prop_h(g1, edges, zeros_h)

    gy = pl.pallas_call(
        _tc_jk_body,
        grid=(grid,),
        in_specs=[degp_spec, s_spec_h, row_spec_h, full((1, hid)),
                  row_spec_h, full((hid, ncls)), full((hid, ncls))],
        out_specs=row_spec_c,
        out_shape=jax.ShapeDtypeStruct((n, ncls), jnp.float32),
    )(degp, s1, g1, b2r, x1, wl1, wl2)

    s2 = prop_c(gy, edges, zeros_c)

    out = pl.pallas_call(
        _tc_out_body,
        grid=(grid,),
        in_specs=[degp_spec, s_spec_c, row_spec_c, full((1, ncls))],
        out_specs=row_spec_c,
        out_shape=jax.ShapeDtypeStruct((n, ncls), jnp.float32),
    )(degp, s2, gy, blinr)

    return (out, out)


# speedup vs baseline: 2.3413x; 2.1795x over previous
import functools

import jax
import jax.numpy as jnp
from jax import lax
from jax.experimental import pallas as pl
from jax.experimental.pallas import tpu as pltpu
from jax.experimental.pallas import tpu_sc as plsc

NC = 2
NS = 16
NW = NC * NS
CH = 128
K0 = 80
K1 = 80
KMAX = max(K0, K1)
K_ALLOC = max(16 * (K0 + K1), 16 * K0 + 15 * K1 + KMAX, 15 * K0 + KMAX)


def _zero_copy_chunks(rows_per_tile):
    chunks = []
    r = 0
    while r < rows_per_tile:
        sz = min(CH, rows_per_tile - r)
        chunks.append((r, sz))
        r += sz
    return chunks


@functools.lru_cache(maxsize=None)
def _sc_propagate(n_pad, d, with_gather):
    rows_per_tile = n_pad // NS
    chunks = _zero_copy_chunks(rows_per_tile)
    mesh = plsc.VectorSubcoreMesh(core_axis_name="c", subcore_axis_name="s")
    k_deg = (16 * (K0 + K1)) // NW

    def zero_acc(const_hbm, buf_v, acc_sh, r0):
        pltpu.sync_copy(const_hbm.at[pl.ds(0, CH)], buf_v)
        for (off, sz) in chunks:
            pltpu.sync_copy(buf_v.at[pl.ds(0, sz)],
                            acc_sh.at[pl.ds(r0 + off, sz)])

    def write_back(out_hbm, buf_v, acc_sh, cid, r0):
        for (off, sz) in chunks:
            pltpu.sync_copy(acc_sh.at[pl.ds(r0 + off, sz)],
                            buf_v.at[pl.ds(0, sz)])
            pltpu.sync_copy(buf_v.at[pl.ds(0, sz)],
                            out_hbm.at[cid, pl.ds(r0 + off, sz)])

    def gather_body(g_hbm, edges_hbm, const_hbm, out_hbm,
                    idx_v, rows_v, acc_sh, sem):
        cid = lax.axis_index("c")
        sid = lax.axis_index("s")
        r0 = sid * rows_per_tile
        zero_acc(const_hbm, rows_v, acc_sh, r0)
        plsc.subcore_barrier()
        base = jnp.where(cid == 0, sid * K0, NS * K0 + sid * K1)
        k_c = jnp.where(cid == 0, K0, K1)
        pltpu.sync_copy(edges_hbm.at[pl.ds(base, KMAX)], idx_v)

        @pl.loop(0, k_c)
        def _edge_chunk(j):
            pltpu.async_copy(g_hbm.at[idx_v.at[j, 0]], rows_v, sem).wait()
            pltpu.sync_copy(rows_v, acc_sh.at[idx_v.at[j, 1]], add=True)

        plsc.subcore_barrier()
        write_back(out_hbm, rows_v, acc_sh, cid, r0)

    def deg_body(g_hbm, edges_hbm, const_hbm, out_hbm,
                 idx_v, ones_v, acc_sh):
        cid = lax.axis_index("c")
        sid = lax.axis_index("s")
        wid = cid * NS + sid
        r0 = sid * rows_per_tile
        zero_acc(const_hbm, ones_v, acc_sh, r0)
        plsc.subcore_barrier()
        pltpu.sync_copy(edges_hbm.at[pl.ds(wid * k_deg, k_deg)], idx_v)
        pltpu.sync_copy(const_hbm.at[pl.ds(CH, CH)], ones_v)

        @pl.loop(0, k_deg)
        def _edge_chunk(j):
            pltpu.sync_copy(ones_v, acc_sh.at[idx_v.at[j, 1]], add=True)

        plsc.subcore_barrier()
        write_back(out_hbm, ones_v, acc_sh, cid, r0)

    if with_gather:
        scratch = [
            pltpu.VMEM((KMAX, 2, CH), jnp.int32),
            pltpu.VMEM((CH, d), jnp.float32),
            pltpu.VMEM_SHARED((n_pad, d), jnp.float32),
            pltpu.SemaphoreType.DMA,
        ]
        body = gather_body
    else:
        scratch = [
            pltpu.VMEM((k_deg, 2, CH), jnp.int32),
            pltpu.VMEM((CH, d), jnp.float32),
            pltpu.VMEM_SHARED((n_pad, d), jnp.float32),
        ]
        body = deg_body

    return pl.kernel(
        body,
        out_type=jax.ShapeDtypeStruct((NC, n_pad, d), jnp.float32),
        mesh=mesh,
        scratch_types=scratch,
        compiler_params=pltpu.CompilerParams(use_tc_tiling_on_sc=False),
        name=f"sc_prop_d{d}_{'gather' if with_gather else 'deg'}",
    )


def _dinv(degp_ref):
    deg = degp_ref[0, :, 0:1] + degp_ref[1, :, 0:1] + 1.0
    return lax.rsqrt(deg)


def _tc_pre_body(degp, x, w1, g0):
    dinv = _dinv(degp)
    g0[...] = jnp.dot(x[...], w1[...],
                      preferred_element_type=jnp.float32) * dinv


def _tc_mid_body(degp, s, g, w2, b1, x1_out, g1_out):
    dinv = _dinv(degp)
    x1 = jnp.maximum(dinv * (s[0] + s[1] + g[...]) + b1[...], 0.0)
    x1_out[...] = x1
    g1_out[...] = jnp.dot(x1, w2[...],
                          preferred_element_type=jnp.float32) * dinv


def _tc_jk_body(degp, s, g1, b2, x1, wl1, wl2, gy_out):
    dinv = _dinv(degp)
    x2 = jnp.maximum(dinv * (s[0] + s[1] + g1[...]) + b2[...], 0.0)
    y = (jnp.dot(x1[...], wl1[...], preferred_element_type=jnp.float32)
         + jnp.dot(x2, wl2[...], preferred_element_type=jnp.float32))
    gy_out[...] = y * dinv


def _tc_out_body(degp, s, gy, blin, out):
    dinv = _dinv(degp)
    out[...] = dinv * (s[0] + s[1] + gy[...]) + blin[...]


def kernel(x, edge_index, W1, b1, W2, b2, Wlin, blin):
    n, f = x.shape
    hid = W1.shape[1]
    ncls = Wlin.shape[1]
    e = edge_index.shape[1]
    n_pad = -(-(n + CH) // (NS * 8)) * (NS * 8)
    assert 16 * (K0 + K1) * CH >= e or (K0 + K1) < 32
    e_alloc = max(K_ALLOC, -(-e // CH)) * CH
    rb = 1000
    assert n % rb == 0
    grid = n // rb

    pad = e_alloc - e
    pad_lane = jnp.arange(pad, dtype=jnp.int32) % CH
    src_p = jnp.concatenate([edge_index[0], pad_lane])
    dst_p = jnp.concatenate([edge_index[1], n + pad_lane])
    edges = jnp.stack([src_p.reshape(e_alloc // CH, CH),
                       dst_p.reshape(e_alloc // CH, CH)], axis=1)

    zeros_h = jnp.zeros((2 * CH, hid), jnp.float32)
    zeros_c = jnp.zeros((2 * CH, ncls), jnp.float32)
    ones16 = jnp.concatenate([jnp.zeros((CH, 16), jnp.float32),
                              jnp.ones((CH, 16), jnp.float32)])
    b1r = b1.reshape(1, hid)
    b2r = b2.reshape(1, hid)
    blinr = blin.reshape(1, ncls)
    wl1 = Wlin[:hid]
    wl2 = Wlin[hid:]

    deg_kernel = _sc_propagate(n_pad, 16, False)
    prop_h = _sc_propagate(n_pad, hid, True)
    prop_c = _sc_propagate(n_pad, ncls, True)

    dummy16 = jnp.zeros((n, 16), jnp.float32)
    degp = deg_kernel(dummy16, edges, ones16)

    degp_spec = pl.BlockSpec((NC, rb, 16), lambda i: (0, i, 0))
    row_spec_h = pl.BlockSpec((rb, hid), lambda i: (i, 0))
    row_spec_c = pl.BlockSpec((rb, ncls), lambda i: (i, 0))
    s_spec_h = pl.BlockSpec((NC, rb, hid), lambda i: (0, i, 0))
    s_spec_c = pl.BlockSpec((NC, rb, ncls), lambda i: (0, i, 0))
    full = lambda shape: pl.BlockSpec(shape, lambda i: tuple(0 for _ in shape))

    g0 = pl.pallas_call(
        _tc_pre_body,
        grid=(grid,),
        in_specs=[degp_spec, pl.BlockSpec((rb, f), lambda i: (i, 0)),
                  full((f, hid))],
        out_specs=row_spec_h,
        out_shape=jax.ShapeDtypeStruct((n, hid), jnp.float32),
    )(degp, x, W1)

    s0 = prop_h(g0, edges, zeros_h)

    x1, g1 = pl.pallas_call(
        _tc_mid_body,
        grid=(grid,),
        in_specs=[degp_spec, s_spec_h, row_spec_h, full((hid, hid)),
                  full((1, hid))],
        out_specs=[row_spec_h, row_spec_h],
        out_shape=[jax.ShapeDtypeStruct((n, hid), jnp.float32),
                   jax.ShapeDtypeStruct((n, hid), jnp.float32)],
    )(degp, s0, g0, W2, b1r)

    s1 = prop_h(g1, edges, zeros_h)

    gy = pl.pallas_call(
        _tc_jk_body,
        grid=(grid,),
        in_specs=[degp_spec, s_spec_h, row_spec_h, full((1, hid)),
                  row_spec_h, full((hid, ncls)), full((hid, ncls))],
        out_specs=row_spec_c,
        out_shape=jax.ShapeDtypeStruct((n, ncls), jnp.float32),
    )(degp, s1, g1, b2r, x1, wl1, wl2)

    s2 = prop_c(gy, edges, zeros_c)

    out = pl.pallas_call(
        _tc_out_body,
        grid=(grid,),
        in_specs=[degp_spec, s_spec_c, row_spec_c, full((1, ncls))],
        out_specs=row_spec_c,
        out_shape=jax.ShapeDtypeStruct((n, ncls), jnp.float32),
    )(degp, s2, gy, blinr)

    return (out, out)


# speedup vs baseline: 2.5610x; 1.0938x over previous
import functools

import jax
import jax.numpy as jnp
from jax import lax
from jax.experimental import pallas as pl
from jax.experimental.pallas import tpu as pltpu
from jax.experimental.pallas import tpu_sc as plsc

NC = 2
NS = 16
NW = NC * NS
CH = 104
K0 = 98
K1 = 98
KMAX = max(K0, K1)
K_ALLOC = max(16 * (K0 + K1), 16 * K0 + 15 * K1 + KMAX, 15 * K0 + KMAX)


def _zero_copy_chunks(rows_per_tile):
    chunks = []
    r = 0
    while r < rows_per_tile:
        sz = min(CH, rows_per_tile - r)
        chunks.append((r, sz))
        r += sz
    return chunks


@functools.lru_cache(maxsize=None)
def _sc_propagate(n_pad, d, with_gather):
    rows_per_tile = n_pad // NS
    chunks = _zero_copy_chunks(rows_per_tile)
    mesh = plsc.VectorSubcoreMesh(core_axis_name="c", subcore_axis_name="s")
    k_deg = (16 * (K0 + K1)) // NW

    def zero_acc(const_hbm, buf_v, acc_sh, r0):
        pltpu.sync_copy(const_hbm.at[pl.ds(0, CH)], buf_v)
        for (off, sz) in chunks:
            pltpu.sync_copy(buf_v.at[pl.ds(0, sz)],
                            acc_sh.at[pl.ds(r0 + off, sz)])

    def write_back(out_hbm, buf_v, acc_sh, cid, r0):
        for (off, sz) in chunks:
            pltpu.sync_copy(acc_sh.at[pl.ds(r0 + off, sz)],
                            buf_v.at[pl.ds(0, sz)])
            pltpu.sync_copy(buf_v.at[pl.ds(0, sz)],
                            out_hbm.at[cid, pl.ds(r0 + off, sz)])

    def gather_body(g_hbm, edges_hbm, const_hbm, out_hbm,
                    idx_v, rows0, rows1, acc_sh, sem0, sem1):
        cid = lax.axis_index("c")
        sid = lax.axis_index("s")
        r0 = sid * rows_per_tile
        zero_acc(const_hbm, rows0, acc_sh, r0)
        plsc.subcore_barrier()
        base = jnp.where(cid == 0, sid * K0, NS * K0 + sid * K1)
        k_c = jnp.where(cid == 0, K0, K1)
        pltpu.sync_copy(edges_hbm.at[pl.ds(base, KMAX)], idx_v)

        @pl.loop(0, k_c, step=2)
        def _edge_pair(j):
            gd0 = pltpu.async_copy(g_hbm.at[idx_v.at[j, 0]], rows0, sem0)
            gd1 = pltpu.async_copy(g_hbm.at[idx_v.at[j + 1, 0]], rows1, sem1)
            gd0.wait()
            pltpu.sync_copy(rows0, acc_sh.at[idx_v.at[j, 1]], add=True)
            gd1.wait()
            pltpu.sync_copy(rows1, acc_sh.at[idx_v.at[j + 1, 1]], add=True)

        plsc.subcore_barrier()
        write_back(out_hbm, rows0, acc_sh, cid, r0)

    def deg_body(g_hbm, edges_hbm, const_hbm, out_hbm,
                 idx_v, ones_v, acc_sh):
        cid = lax.axis_index("c")
        sid = lax.axis_index("s")
        wid = cid * NS + sid
        r0 = sid * rows_per_tile
        zero_acc(const_hbm, ones_v, acc_sh, r0)
        plsc.subcore_barrier()
        pltpu.sync_copy(edges_hbm.at[pl.ds(wid * k_deg, k_deg)], idx_v)
        pltpu.sync_copy(const_hbm.at[pl.ds(CH, CH)], ones_v)

        @pl.loop(0, k_deg)
        def _edge_chunk(j):
            pltpu.sync_copy(ones_v, acc_sh.at[idx_v.at[j, 1]], add=True)

        plsc.subcore_barrier()
        write_back(out_hbm, ones_v, acc_sh, cid, r0)

    if with_gather:
        scratch = [
            pltpu.VMEM((KMAX, 2, CH), jnp.int32),
            pltpu.VMEM((CH, d), jnp.float32),
            pltpu.VMEM((CH, d), jnp.float32),
            pltpu.VMEM_SHARED((n_pad, d), jnp.float32),
            pltpu.SemaphoreType.DMA,
            pltpu.SemaphoreType.DMA,
        ]
        body = gather_body
    else:
        scratch = [
            pltpu.VMEM((k_deg, 2, CH), jnp.int32),
            pltpu.VMEM((CH, d), jnp.float32),
            pltpu.VMEM_SHARED((n_pad, d), jnp.float32),
        ]
        body = deg_body

    return pl.kernel(
        body,
        out_type=jax.ShapeDtypeStruct((NC, n_pad, d), jnp.float32),
        mesh=mesh,
        scratch_types=scratch,
        compiler_params=pltpu.CompilerParams(use_tc_tiling_on_sc=False),
        name=f"sc_prop_d{d}_{'gather' if with_gather else 'deg'}",
    )


def _dinv(degp_ref):
    deg = degp_ref[0, :, 0:1] + degp_ref[1, :, 0:1] + 1.0
    return lax.rsqrt(deg)


def _tc_pre_body(degp, x, w1, g0):
    dinv = _dinv(degp)
    g0[...] = jnp.dot(x[...], w1[...],
                      preferred_element_type=jnp.float32) * dinv


def _tc_mid_body(degp, s, g, w2, b1, x1_out, g1_out):
    dinv = _dinv(degp)
    x1 = jnp.maximum(dinv * (s[0] + s[1] + g[...]) + b1[...], 0.0)
    x1_out[...] = x1
    g1_out[...] = jnp.dot(x1, w2[...],
                          preferred_element_type=jnp.float32) * dinv


def _tc_jk_body(degp, s, g1, b2, x1, wl1, wl2, gy_out):
    dinv = _dinv(degp)
    x2 = jnp.maximum(dinv * (s[0] + s[1] + g1[...]) + b2[...], 0.0)
    y = (jnp.dot(x1[...], wl1[...], preferred_element_type=jnp.float32)
         + jnp.dot(x2, wl2[...], preferred_element_type=jnp.float32))
    gy_out[...] = y * dinv


def _tc_out_body(degp, s, gy, blin, out):
    dinv = _dinv(degp)
    out[...] = dinv * (s[0] + s[1] + gy[...]) + blin[...]


def kernel(x, edge_index, W1, b1, W2, b2, Wlin, blin):
    n, f = x.shape
    hid = W1.shape[1]
    ncls = Wlin.shape[1]
    e = edge_index.shape[1]
    n_pad = -(-(n + CH) // (NS * 8)) * (NS * 8)
    assert 16 * (K0 + K1) * CH >= e or (K0 + K1) < 32
    e_alloc = max(K_ALLOC, -(-e // CH)) * CH
    rb = 1000
    assert n % rb == 0
    grid = n // rb

    pad = e_alloc - e
    pad_lane = jnp.arange(pad, dtype=jnp.int32) % CH
    src_p = jnp.concatenate([edge_index[0], pad_lane])
    dst_p = jnp.concatenate([edge_index[1], n + pad_lane])
    edges = jnp.stack([src_p.reshape(e_alloc // CH, CH),
                       dst_p.reshape(e_alloc // CH, CH)], axis=1)

    zeros_h = jnp.zeros((2 * CH, hid), jnp.float32)
    zeros_c = jnp.zeros((2 * CH, ncls), jnp.float32)
    ones16 = jnp.concatenate([jnp.zeros((CH, 16), jnp.float32),
                              jnp.ones((CH, 16), jnp.float32)])
    b1r = b1.reshape(1, hid)
    b2r = b2.reshape(1, hid)
    blinr = blin.reshape(1, ncls)
    wl1 = Wlin[:hid]
    wl2 = Wlin[hid:]

    deg_kernel = _sc_propagate(n_pad, 16, False)
    prop_h = _sc_propagate(n_pad, hid, True)
    prop_c = _sc_propagate(n_pad, ncls, True)

    dummy16 = jnp.zeros((n, 16), jnp.float32)
    degp = deg_kernel(dummy16, edges, ones16)

    degp_spec = pl.BlockSpec((NC, rb, 16), lambda i: (0, i, 0))
    row_spec_h = pl.BlockSpec((rb, hid), lambda i: (i, 0))
    row_spec_c = pl.BlockSpec((rb, ncls), lambda i: (i, 0))
    s_spec_h = pl.BlockSpec((NC, rb, hid), lambda i: (0, i, 0))
    s_spec_c = pl.BlockSpec((NC, rb, ncls), lambda i: (0, i, 0))
    full = lambda shape: pl.BlockSpec(shape, lambda i: tuple(0 for _ in shape))

    g0 = pl.pallas_call(
        _tc_pre_body,
        grid=(grid,),
        in_specs=[degp_spec, pl.BlockSpec((rb, f), lambda i: (i, 0)),
                  full((f, hid))],
        out_specs=row_spec_h,
        out_shape=jax.ShapeDtypeStruct((n, hid), jnp.float32),
    )(degp, x, W1)

    s0 = prop_h(g0, edges, zeros_h)

    x1, g1 = pl.pallas_call(
        _tc_mid_body,
        grid=(grid,),
        in_specs=[degp_spec, s_spec_h, row_spec_h, full((hid, hid)),
                  full((1, hid))],
        out_specs=[row_spec_h, row_spec_h],
        out_shape=[jax.ShapeDtypeStruct((n, hid), jnp.float32),
                   jax.ShapeDtypeStruct((n, hid), jnp.float32)],
    )(degp, s0, g0, W2, b1r)

    s1 = prop_h(g1, edges, zeros_h)

    gy = pl.pallas_call(
        _tc_jk_body,
        grid=(grid,),
        in_specs=[degp_spec, s_spec_h, row_spec_h, full((1, hid)),
                  row_spec_h, full((hid, ncls)), full((hid, ncls))],
        out_specs=row_spec_c,
        out_shape=jax.ShapeDtypeStruct((n, ncls), jnp.float32),
    )(degp, s1, g1, b2r, x1, wl1, wl2)

    s2 = prop_c(gy, edges, zeros_c)

    out = pl.pallas_call(
        _tc_out_body,
        grid=(grid,),
        in_specs=[degp_spec, s_spec_c, row_spec_c, full((1, ncls))],
        out_specs=row_spec_c,
        out_shape=jax.ShapeDtypeStruct((n, ncls), jnp.float32),
    )(degp, s2, gy, blinr)

    return (out, out)


# speedup vs baseline: 2.6400x; 1.0309x over previous
import functools

import jax
import jax.numpy as jnp
from jax import lax
from jax.experimental import pallas as pl
from jax.experimental.pallas import tpu as pltpu
from jax.experimental.pallas import tpu_sc as plsc

NC = 2
NS = 16
NW = NC * NS
CH = 104
K0 = 98
K1 = 98
KMAX = max(K0, K1)
K_ALLOC = max(16 * (K0 + K1), 16 * K0 + 15 * K1 + KMAX, 15 * K0 + KMAX)


def _zero_copy_chunks(rows_per_tile):
    chunks = []
    r = 0
    while r < rows_per_tile:
        sz = min(CH, rows_per_tile - r)
        chunks.append((r, sz))
        r += sz
    return chunks


@functools.lru_cache(maxsize=None)
def _sc_propagate(n_pad, d, with_gather):
    rows_per_tile = n_pad // NS
    chunks = _zero_copy_chunks(rows_per_tile)
    mesh = plsc.VectorSubcoreMesh(core_axis_name="c", subcore_axis_name="s")
    k_deg = (16 * (K0 + K1)) // NW

    def zero_acc(const_hbm, buf_v, acc_sh, r0):
        pltpu.sync_copy(const_hbm.at[pl.ds(0, CH)], buf_v)
        for (off, sz) in chunks:
            pltpu.sync_copy(buf_v.at[pl.ds(0, sz)],
                            acc_sh.at[pl.ds(r0 + off, sz)])

    def write_back(out_hbm, buf_v, acc_sh, cid, r0):
        for (off, sz) in chunks:
            pltpu.sync_copy(acc_sh.at[pl.ds(r0 + off, sz)],
                            buf_v.at[pl.ds(0, sz)])
            pltpu.sync_copy(buf_v.at[pl.ds(0, sz)],
                            out_hbm.at[cid, pl.ds(r0 + off, sz)])

    def gather_body(g_hbm, src_hbm, dst_hbm, const_hbm, out_hbm,
                    src_v, dst_v, rows0, rows1, acc_sh, sem0, sem1):
        cid = lax.axis_index("c")
        sid = lax.axis_index("s")
        r0 = sid * rows_per_tile
        zero_acc(const_hbm, rows0, acc_sh, r0)
        plsc.subcore_barrier()
        base = jnp.where(cid == 0, sid * K0, NS * K0 + sid * K1)
        k_c = jnp.where(cid == 0, K0, K1)
        pltpu.sync_copy(src_hbm.at[pl.ds(base, KMAX)], src_v)
        pltpu.sync_copy(dst_hbm.at[pl.ds(base, KMAX)], dst_v)

        @pl.loop(0, k_c, step=2)
        def _edge_pair(j):
            gd0 = pltpu.async_copy(g_hbm.at[src_v.at[j]], rows0, sem0)
            gd1 = pltpu.async_copy(g_hbm.at[src_v.at[j + 1]], rows1, sem1)
            gd0.wait()
            pltpu.sync_copy(rows0, acc_sh.at[dst_v.at[j]], add=True)
            gd1.wait()
            pltpu.sync_copy(rows1, acc_sh.at[dst_v.at[j + 1]], add=True)

        plsc.subcore_barrier()
        write_back(out_hbm, rows0, acc_sh, cid, r0)

    def deg_body(g_hbm, src_hbm, dst_hbm, const_hbm, out_hbm,
                 idx_v, ones_v, acc_sh):
        cid = lax.axis_index("c")
        sid = lax.axis_index("s")
        wid = cid * NS + sid
        r0 = sid * rows_per_tile
        zero_acc(const_hbm, ones_v, acc_sh, r0)
        plsc.subcore_barrier()
        pltpu.sync_copy(dst_hbm.at[pl.ds(wid * k_deg, k_deg)], idx_v)
        pltpu.sync_copy(const_hbm.at[pl.ds(CH, CH)], ones_v)

        @pl.loop(0, k_deg)
        def _edge_chunk(j):
            pltpu.sync_copy(ones_v, acc_sh.at[idx_v.at[j]], add=True)

        plsc.subcore_barrier()
        write_back(out_hbm, ones_v, acc_sh, cid, r0)

    if with_gather:
        scratch = [
            pltpu.VMEM((KMAX, CH), jnp.int32),
            pltpu.VMEM((KMAX, CH), jnp.int32),
            pltpu.VMEM((CH, d), jnp.float32),
            pltpu.VMEM((CH, d), jnp.float32),
            pltpu.VMEM_SHARED((n_pad, d), jnp.float32),
            pltpu.SemaphoreType.DMA,
            pltpu.SemaphoreType.DMA,
        ]
        body = gather_body
    else:
        scratch = [
            pltpu.VMEM((k_deg, CH), jnp.int32),
            pltpu.VMEM((CH, d), jnp.float32),
            pltpu.VMEM_SHARED((n_pad, d), jnp.float32),
        ]
        body = deg_body

    return pl.kernel(
        body,
        out_type=jax.ShapeDtypeStruct((NC, n_pad, d), jnp.float32),
        mesh=mesh,
        scratch_types=scratch,
        compiler_params=pltpu.CompilerParams(use_tc_tiling_on_sc=False),
        name=f"sc_prop_d{d}_{'gather' if with_gather else 'deg'}",
    )


def _dinv(degp_ref):
    deg = degp_ref[0, :, 0:1] + degp_ref[1, :, 0:1] + 1.0
    return lax.rsqrt(deg)


def _tc_pre_body(degp, x, w1, g0):
    dinv = _dinv(degp)
    g0[...] = jnp.dot(x[...], w1[...],
                      preferred_element_type=jnp.float32) * dinv


def _tc_mid_body(degp, s, g, w2, b1, x1_out, g1_out):
    dinv = _dinv(degp)
    x1 = jnp.maximum(dinv * (s[0] + s[1] + g[...]) + b1[...], 0.0)
    x1_out[...] = x1
    g1_out[...] = jnp.dot(x1, w2[...],
                          preferred_element_type=jnp.float32) * dinv


def _tc_jk_body(degp, s, g1, b2, x1, wl1, wl2, gy_out):
    dinv = _dinv(degp)
    x2 = jnp.maximum(dinv * (s[0] + s[1] + g1[...]) + b2[...], 0.0)
    y = (jnp.dot(x1[...], wl1[...], preferred_element_type=jnp.float32)
         + jnp.dot(x2, wl2[...], preferred_element_type=jnp.float32))
    gy_out[...] = y * dinv


def _tc_out_body(degp, s, gy, blin, out):
    dinv = _dinv(degp)
    out[...] = dinv * (s[0] + s[1] + gy[...]) + blin[...]


def kernel(x, edge_index, W1, b1, W2, b2, Wlin, blin):
    n, f = x.shape
    hid = W1.shape[1]
    ncls = Wlin.shape[1]
    e = edge_index.shape[1]
    n_pad = -(-(n + CH) // (NS * 8)) * (NS * 8)
    assert 16 * (K0 + K1) * CH >= e or (K0 + K1) < 32
    e_alloc = max(K_ALLOC, -(-e // CH)) * CH
    rb = 1000
    assert n % rb == 0
    grid = n // rb

    pad = e_alloc - e
    pad_lane = jnp.arange(pad, dtype=jnp.int32) % CH
    src_p = jnp.concatenate([edge_index[0], pad_lane])
    dst_p = jnp.concatenate([edge_index[1], n + pad_lane])
    src2 = src_p.reshape(e_alloc // CH, CH)
    dst2 = dst_p.reshape(e_alloc // CH, CH)

    zeros_h = jnp.zeros((2 * CH, hid), jnp.float32)
    zeros_c = jnp.zeros((2 * CH, ncls), jnp.float32)
    ones16 = jnp.concatenate([jnp.zeros((CH, 16), jnp.float32),
                              jnp.ones((CH, 16), jnp.float32)])
    b1r = b1.reshape(1, hid)
    b2r = b2.reshape(1, hid)
    blinr = blin.reshape(1, ncls)
    wl1 = Wlin[:hid]
    wl2 = Wlin[hid:]

    deg_kernel = _sc_propagate(n_pad, 16, False)
    prop_h = _sc_propagate(n_pad, hid, True)
    prop_c = _sc_propagate(n_pad, ncls, True)

    dummy16 = jnp.zeros((n, 16), jnp.float32)
    degp = deg_kernel(dummy16, src2, dst2, ones16)

    degp_spec = pl.BlockSpec((NC, rb, 16), lambda i: (0, i, 0))
    row_spec_h = pl.BlockSpec((rb, hid), lambda i: (i, 0))
    row_spec_c = pl.BlockSpec((rb, ncls), lambda i: (i, 0))
    s_spec_h = pl.BlockSpec((NC, rb, hid), lambda i: (0, i, 0))
    s_spec_c = pl.BlockSpec((NC, rb, ncls), lambda i: (0, i, 0))
    full = lambda shape: pl.BlockSpec(shape, lambda i: tuple(0 for _ in shape))

    g0 = pl.pallas_call(
        _tc_pre_body,
        grid=(grid,),
        in_specs=[degp_spec, pl.BlockSpec((rb, f), lambda i: (i, 0)),
                  full((f, hid))],
        out_specs=row_spec_h,
        out_shape=jax.ShapeDtypeStruct((n, hid), jnp.float32),
    )(degp, x, W1)

    s0 = prop_h(g0, src2, dst2, zeros_h)

    x1, g1 = pl.pallas_call(
        _tc_mid_body,
        grid=(grid,),
        in_specs=[degp_spec, s_spec_h, row_spec_h, full((hid, hid)),
                  full((1, hid))],
        out_specs=[row_spec_h, row_spec_h],
        out_shape=[jax.ShapeDtypeStruct((n, hid), jnp.float32),
                   jax.ShapeDtypeStruct((n, hid), jnp.float32)],
    )(degp, s0, g0, W2, b1r)

    s1 = prop_h(g1, src2, dst2, zeros_h)

    gy = pl.pallas_call(
        _tc_jk_body,
        grid=(grid,),
        in_specs=[degp_spec, s_spec_h, row_spec_h, full((1, hid)),
                  row_spec_h, full((hid, ncls)), full((hid, ncls))],
        out_specs=row_spec_c,
        out_shape=jax.ShapeDtypeStruct((n, ncls), jnp.float32),
    )(degp, s1, g1, b2r, x1, wl1, wl2)

    s2 = prop_c(gy, src2, dst2, zeros_c)

    out = pl.pallas_call(
        _tc_out_body,
        grid=(grid,),
        in_specs=[degp_spec, s_spec_c, row_spec_c, full((1, ncls))],
        out_specs=row_spec_c,
        out_shape=jax.ShapeDtypeStruct((n, ncls), jnp.float32),
    )(degp, s2, gy, blinr)

    return (out, out)


# speedup vs baseline: 2.6532x; 1.0050x over previous
import functools

import jax
import jax.numpy as jnp
from jax import lax
from jax.experimental import pallas as pl
from jax.experimental.pallas import tpu as pltpu
from jax.experimental.pallas import tpu_sc as plsc

NC = 2
NS = 16
NW = NC * NS
CH = 128
KPT = 80
IB = 40


def _zero_copy_chunks(rows_per_tile):
    chunks = []
    r = 0
    while r < rows_per_tile:
        sz = min(CH, rows_per_tile - r)
        chunks.append((r, sz))
        r += sz
    return chunks


@functools.lru_cache(maxsize=None)
def _sc_propagate(n_pad, d, with_gather):
    rows_per_tile = n_pad // NS
    chunks = _zero_copy_chunks(rows_per_tile)
    mesh = plsc.VectorSubcoreMesh(core_axis_name="c", subcore_axis_name="s")

    def zero_acc(const_hbm, buf_v, acc_sh, r0):
        pltpu.sync_copy(const_hbm.at[pl.ds(0, CH)], buf_v)
        for (off, sz) in chunks:
            pltpu.sync_copy(buf_v.at[pl.ds(0, sz)],
                            acc_sh.at[pl.ds(r0 + off, sz)])

    def write_back(out_hbm, buf_v, acc_sh, cid, r0):
        for (off, sz) in chunks:
            pltpu.sync_copy(acc_sh.at[pl.ds(r0 + off, sz)],
                            buf_v.at[pl.ds(0, sz)])
            pltpu.sync_copy(buf_v.at[pl.ds(0, sz)],
                            out_hbm.at[cid, pl.ds(r0 + off, sz)])

    def gather_body(g_hbm, src_hbm, dst_hbm, const_hbm, out_hbm,
                    src_v, dst_v, rows0, rows1, acc_sh, sem0, sem1):
        cid = lax.axis_index("c")
        sid = lax.axis_index("s")
        r0 = sid * rows_per_tile
        wid = cid * NS + sid
        zero_acc(const_hbm, rows0, acc_sh, r0)
        plsc.subcore_barrier()
        for b in range(KPT // IB):
            pltpu.sync_copy(src_hbm.at[wid, pl.ds(b * IB, IB)], src_v)
            pltpu.sync_copy(dst_hbm.at[wid, pl.ds(b * IB, IB)], dst_v)

            @pl.loop(0, IB, step=2)
            def _edge_pair(j):
                gd0 = pltpu.async_copy(g_hbm.at[src_v.at[j]], rows0, sem0)
                gd1 = pltpu.async_copy(
                    g_hbm.at[src_v.at[j + 1]], rows1, sem1)
                gd0.wait()
                pltpu.sync_copy(rows0, acc_sh.at[dst_v.at[j]], add=True)
                gd1.wait()
                pltpu.sync_copy(rows1, acc_sh.at[dst_v.at[j + 1]], add=True)

        plsc.subcore_barrier()
        write_back(out_hbm, rows0, acc_sh, cid, r0)

    def deg_body(g_hbm, src_hbm, dst_hbm, const_hbm, out_hbm,
                 idx_v, ones_v, acc_sh):
        cid = lax.axis_index("c")
        sid = lax.axis_index("s")
        wid = cid * NS + sid
        r0 = sid * rows_per_tile
        zero_acc(const_hbm, ones_v, acc_sh, r0)
        plsc.subcore_barrier()
        pltpu.sync_copy(dst_hbm.at[wid], idx_v)
        pltpu.sync_copy(const_hbm.at[pl.ds(CH, CH)], ones_v)

        @pl.loop(0, KPT)
        def _edge_chunk(j):
            pltpu.sync_copy(ones_v, acc_sh.at[idx_v.at[j]], add=True)

        plsc.subcore_barrier()
        write_back(out_hbm, ones_v, acc_sh, cid, r0)

    if with_gather:
        scratch = [
            pltpu.VMEM((IB, CH), jnp.int32),
            pltpu.VMEM((IB, CH), jnp.int32),
            pltpu.VMEM((CH, d), jnp.float32),
            pltpu.VMEM((CH, d), jnp.float32),
            pltpu.VMEM_SHARED((n_pad, d), jnp.float32),
            pltpu.SemaphoreType.DMA,
            pltpu.SemaphoreType.DMA,
        ]
        body = gather_body
    else:
        scratch = [
            pltpu.VMEM((KPT, CH), jnp.int32),
            pltpu.VMEM((CH, d), jnp.float32),
            pltpu.VMEM_SHARED((n_pad, d), jnp.float32),
        ]
        body = deg_body

    tiled = (d % 128 == 0) or not with_gather
    return pl.kernel(
        body,
        out_type=jax.ShapeDtypeStruct((NC, n_pad, d), jnp.float32),
        mesh=mesh,
        scratch_types=scratch,
        compiler_params=pltpu.CompilerParams(use_tc_tiling_on_sc=tiled),
        name=f"sc_prop_d{d}_{'gather' if with_gather else 'deg'}",
    )


def _dinv(degp_ref):
    deg = degp_ref[0, :, 0:1] + degp_ref[1, :, 0:1] + 1.0
    return lax.rsqrt(deg)


def _tc_pre_body(degp, x, w1, g0):
    dinv = _dinv(degp)
    g0[...] = jnp.dot(x[...], w1[...],
                      preferred_element_type=jnp.float32) * dinv


def _tc_mid_body(degp, s, g, w2, b1, x1_out, g1_out):
    dinv = _dinv(degp)
    x1 = jnp.maximum(dinv * (s[0] + s[1] + g[...]) + b1[...], 0.0)
    x1_out[...] = x1
    g1_out[...] = jnp.dot(x1, w2[...],
                          preferred_element_type=jnp.float32) * dinv


def _tc_jk_body(degp, s, g1, b2, x1, wl1, wl2, gy_out):
    dinv = _dinv(degp)
    x2 = jnp.maximum(dinv * (s[0] + s[1] + g1[...]) + b2[...], 0.0)
    y = (jnp.dot(x1[...], wl1[...], preferred_element_type=jnp.float32)
         + jnp.dot(x2, wl2[...], preferred_element_type=jnp.float32))
    gy_out[...] = y * dinv


def _tc_out_body(degp, s, gy, blin, out):
    dinv = _dinv(degp)
    out[...] = dinv * (s[0] + s[1] + gy[...]) + blin[...]


def kernel(x, edge_index, W1, b1, W2, b2, Wlin, blin):
    n, f = x.shape
    hid = W1.shape[1]
    ncls = Wlin.shape[1]
    e = edge_index.shape[1]
    n_pad = -(-(n + CH) // (NS * 8)) * (NS * 8)
    assert NW * KPT * CH >= e
    e_alloc = NW * KPT * CH
    rb = 1000
    assert n % rb == 0
    grid = n // rb

    pad = e_alloc - e
    pad_lane = jnp.arange(pad, dtype=jnp.int32) % CH
    src_p = jnp.concatenate([edge_index[0], pad_lane])
    dst_p = jnp.concatenate([edge_index[1], n + pad_lane])
    src2 = src_p.reshape(NW, KPT, CH)
    dst2 = dst_p.reshape(NW, KPT, CH)

    zeros_h = jnp.zeros((2 * CH, hid), jnp.float32)
    zeros_c = jnp.zeros((2 * CH, ncls), jnp.float32)
    ones16 = jnp.concatenate([jnp.zeros((CH, 16), jnp.float32),
                              jnp.ones((CH, 16), jnp.float32)])
    b1r = b1.reshape(1, hid)
    b2r = b2.reshape(1, hid)
    blinr = blin.reshape(1, ncls)
    wl1 = Wlin[:hid]
    wl2 = Wlin[hid:]

    deg_kernel = _sc_propagate(n_pad, 16, False)
    prop_h = _sc_propagate(n_pad, hid, True)
    prop_c = _sc_propagate(n_pad, ncls, True)

    dummy16 = jnp.zeros((n, 16), jnp.float32)
    degp = deg_kernel(dummy16, src2, dst2, ones16)

    degp_spec = pl.BlockSpec((NC, rb, 16), lambda i: (0, i, 0))
    row_spec_h = pl.BlockSpec((rb, hid), lambda i: (i, 0))
    row_spec_c = pl.BlockSpec((rb, ncls), lambda i: (i, 0))
    s_spec_h = pl.BlockSpec((NC, rb, hid), lambda i: (0, i, 0))
    s_spec_c = pl.BlockSpec((NC, rb, ncls), lambda i: (0, i, 0))
    full = lambda shape: pl.BlockSpec(shape, lambda i: tuple(0 for _ in shape))

    g0 = pl.pallas_call(
        _tc_pre_body,
        grid=(grid,),
        in_specs=[degp_spec, pl.BlockSpec((rb, f), lambda i: (i, 0)),
                  full((f, hid))],
        out_specs=row_spec_h,
        out_shape=jax.ShapeDtypeStruct((n, hid), jnp.float32),
    )(degp, x, W1)

    s0 = prop_h(g0, src2, dst2, zeros_h)

    x1, g1 = pl.pallas_call(
        _tc_mid_body,
        grid=(grid,),
        in_specs=[degp_spec, s_spec_h, row_spec_h, full((hid, hid)),
                  full((1, hid))],
        out_specs=[row_spec_h, row_spec_h],
        out_shape=[jax.ShapeDtypeStruct((n, hid), jnp.float32),
                   jax.ShapeDtypeStruct((n, hid), jnp.float32)],
    )(degp, s0, g0, W2, b1r)

    s1 = prop_h(g1, src2, dst2, zeros_h)

    gy = pl.pallas_call(
        _tc_jk_body,
        grid=(grid,),
        in_specs=[degp_spec, s_spec_h, row_spec_h, full((1, hid)),
                  row_spec_h, full((hid, ncls)), full((hid, ncls))],
        out_specs=row_spec_c,
        out_shape=jax.ShapeDtypeStruct((n, ncls), jnp.float32),
    )(degp, s1, g1, b2r, x1, wl1, wl2)

    s2 = prop_c(gy, src2, dst2, zeros_c)

    out = pl.pallas_call(
        _tc_out_body,
        grid=(grid,),
        in_specs=[degp_spec, s_spec_c, row_spec_c, full((1, ncls))],
        out_specs=row_spec_c,
        out_shape=jax.ShapeDtypeStruct((n, ncls), jnp.float32),
    )(degp, s2, gy, blinr)

    return (out, out)
